# baseline (speedup 1.0000x reference)
import functools

import numpy as np
import jax
import jax.numpy as jnp
from jax import lax
from jax.experimental import pallas as pl
from jax.experimental.pallas import tpu as pltpu
from jax.experimental.pallas import tpu_sc as plsc

N = 10000
E = 160000
D = 16
DA = 272
DB = 288

NC = 2
NS = 16
NW = NC * NS
EPW = E // NW
CHUNK = 100
NCHUNK = EPW // CHUNK
NPAD = 10240
RPT = NPAD // NS

MBLK = 1000


def _matmul_body(f_ref, w_ref, g_ref):
    g_ref[...] = jnp.dot(f_ref[...], w_ref[...],
                         preferred_element_type=jnp.float32
                         ).astype(jnp.bfloat16)


_PERM = np.empty((DB,), np.int32)
for _p in range(DB // 32):
    for _j in range(16):
        _PERM[32 * _p + 2 * _j] = 16 * (2 * _p) + _j
        _PERM[32 * _p + 2 * _j + 1] = 16 * (2 * _p + 1) + _j


def _combine_body(p_ref, f_ref, b_ref, o_ref):
    o_ref[...] = p_ref[0, :N] + p_ref[1, :N] + f_ref[...] + b_ref[...]


def _sc_body(g_hbm, ef_hbm, src_hbm, dst_hbm, out_hbm,
             srcall_v, dstall_v, ef_v, rows_v, msg_v, zero_v, acc_sh, sems):
    cid = lax.axis_index("c")
    sid = lax.axis_index("s")
    wid = cid * NS + sid
    wbase = wid * EPW

    pltpu.sync_copy(src_hbm.at[pl.ds(wid * NCHUNK, NCHUNK), :], srcall_v)
    pltpu.sync_copy(dst_hbm.at[pl.ds(wid * NCHUNK, NCHUNK), :], dstall_v)

    def zero_row(r, carry):
        zero_v[r, :] = jnp.zeros((D,), jnp.float32)
        return carry

    lax.fori_loop(0, RPT, zero_row, 0)
    pltpu.sync_copy(zero_v, acc_sh.at[pl.ds(sid * RPT, RPT)])
    plsc.subcore_barrier()

    def issue(c, slot):
        pltpu.async_copy(ef_hbm.at[pl.ds(wbase + c * CHUNK, CHUNK), :],
                         ef_v[slot], sems[slot])
        pltpu.async_copy(g_hbm.at[srcall_v.at[c]], rows_v[slot], sems[slot])

    def wait(slot):
        pltpu.make_async_copy(ef_hbm.at[pl.ds(0, CHUNK), :],
                              ef_v[slot], sems[slot]).wait()
        pltpu.make_async_copy(g_hbm.at[pl.ds(0, CHUNK), :],
                              rows_v[slot], sems[slot]).wait()

    def compute(c, slot):
        erows = rows_v[slot]
        eef = ef_v[slot]

        def edge_body(e, ecarry):
            ef_row = eef[e, :]
            acc, _ = plsc.unpack(erows[e, pl.ds(256, 32)],
                                 format=plsc.PackFormat.INTERLEAVED)
            for p in range(8):
                a, b = plsc.unpack(erows[e, pl.ds(32 * p, 32)],
                                   format=plsc.PackFormat.INTERLEAVED)
                acc = acc + ef_row[2 * p] * a + ef_row[2 * p + 1] * b
            msg_v[e, :] = acc
            return ecarry

        lax.fori_loop(0, CHUNK, edge_body, 0)
        pltpu.sync_copy(msg_v, acc_sh.at[dstall_v.at[c]], add=True)

    issue(0, 0)

    def pair_body(j, carry):
        c0 = 2 * j
        wait(0)
        issue(c0 + 1, 1)
        compute(c0, 0)
        wait(1)
        issue(c0 + 2, 0)
        compute(c0 + 1, 1)
        return carry

    lax.fori_loop(0, (NCHUNK - 2) // 2, pair_body, 0)
    wait(0)
    issue(NCHUNK - 1, 1)
    compute(NCHUNK - 2, 0)
    wait(1)
    compute(NCHUNK - 1, 1)

    plsc.subcore_barrier()
    pltpu.sync_copy(acc_sh.at[pl.ds(sid * RPT, RPT)],
                    out_hbm.at[cid, pl.ds(sid * RPT, RPT), :])


_sc_call = functools.partial(
    pl.kernel,
    out_type=jax.ShapeDtypeStruct((NC, NPAD, D), jnp.float32),
    mesh=plsc.VectorSubcoreMesh(core_axis_name="c", subcore_axis_name="s"),
    scratch_types=[
        pltpu.VMEM((NCHUNK, CHUNK), jnp.int32),
        pltpu.VMEM((NCHUNK, CHUNK), jnp.int32),
        [pltpu.VMEM((CHUNK, D), jnp.float32)] * 2,
        [pltpu.VMEM((CHUNK, DB), jnp.bfloat16)] * 2,
        pltpu.VMEM((CHUNK, D), jnp.float32),
        pltpu.VMEM((RPT, D), jnp.float32),
        pltpu.VMEM_SHARED((NPAD, D), jnp.float32),
        [pltpu.SemaphoreType.DMA] * 2,
    ],
    compiler_params=pltpu.CompilerParams(use_tc_tiling_on_sc=False,
                                         needs_layout_passes=False),
)(_sc_body)


@jax.jit
def kernel(feat, efeat, edge_index, W_edge, b_edge, bias):
    Wr = W_edge.reshape(D, D, D).transpose(1, 0, 2).reshape(D, D * D)
    Waug = jnp.concatenate([Wr, b_edge.reshape(D, D),
                            jnp.zeros((D, DB - DA), jnp.float32)], axis=1)
    Waug = Waug[:, _PERM]

    g_aug = pl.pallas_call(
        _matmul_body,
        grid=(N // MBLK,),
        in_specs=[
            pl.BlockSpec((MBLK, D), lambda i: (i, 0)),
            pl.BlockSpec((D, DB), lambda i: (0, 0)),
        ],
        out_specs=pl.BlockSpec((MBLK, DB), lambda i: (i, 0)),
        out_shape=jax.ShapeDtypeStruct((N, DB), jnp.bfloat16),
    )(feat, Waug)

    src2d = edge_index[0].reshape(NW * NCHUNK, CHUNK)
    dst2d = edge_index[1].reshape(NW * NCHUNK, CHUNK)
    partials = _sc_call(g_aug, efeat, src2d, dst2d)

    out = pl.pallas_call(
        _combine_body,
        out_shape=jax.ShapeDtypeStruct((N, D), jnp.float32),
    )(partials, feat, bias.reshape(1, D))
    return out


# speedup vs baseline: 1.2971x; 1.2971x over previous
import functools

import numpy as np
import jax
import jax.numpy as jnp
from jax import lax
from jax.experimental import pallas as pl
from jax.experimental.pallas import tpu as pltpu
from jax.experimental.pallas import tpu_sc as plsc

N = 10000
E = 160000
D = 16
DA = 272
DB = 288

NC = 2
NS = 16
NW = NC * NS
EPW = E // NW
CHUNK = 200
NCHUNK = EPW // CHUNK
NPAD = 10240
RPT = NPAD // NS

MBLK = 2000


def _matmul_body(f_ref, w_ref, g_ref):
    g_ref[...] = jnp.dot(f_ref[...], w_ref[...],
                         preferred_element_type=jnp.float32
                         ).astype(jnp.bfloat16)


_PERM = np.empty((DB,), np.int32)
for _p in range(DB // 32):
    for _j in range(16):
        _PERM[32 * _p + 2 * _j] = 16 * (2 * _p) + _j
        _PERM[32 * _p + 2 * _j + 1] = 16 * (2 * _p + 1) + _j


def _combine_body(p_ref, f_ref, b_ref, o_ref):
    o_ref[...] = p_ref[0, :N] + p_ref[1, :N] + f_ref[...] + b_ref[...]


def _sc_body(g_hbm, ef_hbm, src_hbm, dst_hbm, out_hbm,
             srcall_v, dstall_v, ef_v, rows_v, msg_v, zero_v, acc_sh, sems):
    cid = lax.axis_index("c")
    sid = lax.axis_index("s")
    wid = cid * NS + sid
    wbase = wid * EPW

    pltpu.sync_copy(src_hbm.at[pl.ds(wbase, EPW)], srcall_v)
    pltpu.sync_copy(dst_hbm.at[pl.ds(wbase, EPW)], dstall_v)

    def zero_row(r, carry):
        zero_v[r, :] = jnp.zeros((D,), jnp.float32)
        return carry

    lax.fori_loop(0, RPT, zero_row, 0)
    pltpu.sync_copy(zero_v, acc_sh.at[pl.ds(sid * RPT, RPT)])
    plsc.subcore_barrier()

    def issue(c, slot):
        pltpu.async_copy(ef_hbm.at[:, pl.ds(wbase + c * CHUNK, CHUNK)],
                         ef_v[slot], sems[slot])
        pltpu.async_copy(g_hbm.at[srcall_v.at[pl.ds(c * CHUNK, CHUNK)]],
                         rows_v[slot], sems[slot])

    def wait(slot):
        pltpu.make_async_copy(ef_hbm.at[:, pl.ds(0, CHUNK)],
                              ef_v[slot], sems[slot]).wait()
        pltpu.make_async_copy(g_hbm.at[pl.ds(0, CHUNK), :],
                              rows_v[slot], sems[slot]).wait()

    def compute(c, slot):
        erows = rows_v[slot]
        eef = ef_v[slot]

        lanes = lax.iota(jnp.int32, D)

        def edge_body(e, ecarry):
            ef_row = plsc.load_gather(eef, [lanes, jnp.full((D,), e, jnp.int32)])
            bias_t, _ = plsc.unpack(erows[e, pl.ds(256, 32)],
                                    format=plsc.PackFormat.INTERLEAVED)
            terms = [bias_t]
            for p in range(8):
                a, b = plsc.unpack(erows[e, pl.ds(32 * p, 32)],
                                   format=plsc.PackFormat.INTERLEAVED)
                terms.append(ef_row[2 * p] * a + ef_row[2 * p + 1] * b)
            while len(terms) > 1:
                terms = [terms[i] + terms[i + 1]
                         for i in range(0, len(terms) - 1, 2)] + (
                             [terms[-1]] if len(terms) % 2 else [])
            msg_v[e, :] = terms[0]
            return ecarry

        lax.fori_loop(0, CHUNK, edge_body, 0)
        pltpu.sync_copy(msg_v, acc_sh.at[dstall_v.at[pl.ds(c * CHUNK, CHUNK)]],
                        add=True)

    issue(0, 0)

    def pair_body(j, carry):
        c0 = 2 * j
        wait(0)
        issue(c0 + 1, 1)
        compute(c0, 0)
        wait(1)
        issue(c0 + 2, 0)
        compute(c0 + 1, 1)
        return carry

    lax.fori_loop(0, (NCHUNK - 1) // 2, pair_body, 0)
    wait(0)
    compute(NCHUNK - 1, 0)

    plsc.subcore_barrier()
    pltpu.sync_copy(acc_sh.at[pl.ds(sid * RPT, RPT)],
                    out_hbm.at[cid, pl.ds(sid * RPT, RPT), :])


_sc_call = functools.partial(
    pl.kernel,
    out_type=jax.ShapeDtypeStruct((NC, NPAD, D), jnp.float32),
    mesh=plsc.VectorSubcoreMesh(core_axis_name="c", subcore_axis_name="s"),
    scratch_types=[
        pltpu.VMEM((EPW,), jnp.int32),
        pltpu.VMEM((EPW,), jnp.int32),
        [pltpu.VMEM((D, CHUNK), jnp.float32)] * 2,
        [pltpu.VMEM((CHUNK, DB), jnp.bfloat16)] * 2,
        pltpu.VMEM((CHUNK, D), jnp.float32),
        pltpu.VMEM((RPT, D), jnp.float32),
        pltpu.VMEM_SHARED((NPAD, D), jnp.float32),
        [pltpu.SemaphoreType.DMA] * 2,
    ],
    compiler_params=pltpu.CompilerParams(use_tc_tiling_on_sc=False,
                                         needs_layout_passes=False),
)(_sc_body)


@jax.jit
def kernel(feat, efeat, edge_index, W_edge, b_edge, bias):
    Wr = W_edge.reshape(D, D, D).transpose(1, 0, 2).reshape(D, D * D)
    Waug = jnp.concatenate([Wr, b_edge.reshape(D, D),
                            jnp.zeros((D, DB - DA), jnp.float32)], axis=1)
    Waug = Waug[:, _PERM]

    g_aug = pl.pallas_call(
        _matmul_body,
        grid=(N // MBLK,),
        in_specs=[
            pl.BlockSpec((MBLK, D), lambda i: (i, 0)),
            pl.BlockSpec((D, DB), lambda i: (0, 0)),
        ],
        out_specs=pl.BlockSpec((MBLK, DB), lambda i: (i, 0)),
        out_shape=jax.ShapeDtypeStruct((N, DB), jnp.bfloat16),
    )(feat, Waug)

    partials = _sc_call(g_aug, efeat.T, edge_index[0], edge_index[1])

    out = pl.pallas_call(
        _combine_body,
        out_shape=jax.ShapeDtypeStruct((N, D), jnp.float32),
    )(partials, feat, bias.reshape(1, D))
    return out


# speedup vs baseline: 1.3179x; 1.0160x over previous
import functools

import numpy as np
import jax
import jax.numpy as jnp
from jax import lax
from jax.experimental import pallas as pl
from jax.experimental.pallas import tpu as pltpu
from jax.experimental.pallas import tpu_sc as plsc

N = 10000
E = 160000
D = 16
DA = 272
DB = 288

NC = 2
NS = 16
NW = NC * NS
EPW = E // NW
CHUNK = 200
NCHUNK = EPW // CHUNK
NPAD = 10240
RPT = NPAD // NS

MBLK = 2000


def _matmul_body(f_ref, w_ref, g_ref):
    g_ref[...] = jnp.dot(f_ref[...], w_ref[...],
                         preferred_element_type=jnp.float32
                         ).astype(jnp.bfloat16)


_PERM = np.empty((DB,), np.int32)
for _p in range(DB // 32):
    for _j in range(16):
        _PERM[32 * _p + 2 * _j] = 16 * (2 * _p) + _j
        _PERM[32 * _p + 2 * _j + 1] = 16 * (2 * _p + 1) + _j


def _combine_body(p_ref, f_ref, b_ref, o_ref):
    o_ref[...] = p_ref[0, :N] + p_ref[1, :N] + f_ref[...] + b_ref[...]


def _sc_body(g_hbm, ef_hbm, src_hbm, dst_hbm, out_hbm,
             srcall_v, dstall_v, ef_v, rows_v, msg_v, zero_v, acc_sh, sems,
             ssems):
    cid = lax.axis_index("c")
    sid = lax.axis_index("s")
    wid = cid * NS + sid
    wbase = wid * EPW

    pltpu.sync_copy(src_hbm.at[pl.ds(wbase, EPW)], srcall_v)
    pltpu.sync_copy(dst_hbm.at[pl.ds(wbase, EPW)], dstall_v)

    def zero_row(r, carry):
        zero_v[r, :] = jnp.zeros((D,), jnp.float32)
        return carry

    lax.fori_loop(0, RPT, zero_row, 0)
    pltpu.sync_copy(zero_v, acc_sh.at[pl.ds(sid * RPT, RPT)])

    def zero_msg(r, carry):
        msg_v[0][r, :] = jnp.zeros((D,), jnp.float32)
        msg_v[1][r, :] = jnp.zeros((D,), jnp.float32)
        return carry

    lax.fori_loop(0, CHUNK, zero_msg, 0)
    plsc.subcore_barrier()

    def issue(c, slot):
        pltpu.async_copy(ef_hbm.at[:, pl.ds(wbase + c * CHUNK, CHUNK)],
                         ef_v[slot], sems[slot])
        pltpu.async_copy(g_hbm.at[srcall_v.at[pl.ds(c * CHUNK, CHUNK)]],
                         rows_v[slot], sems[slot])

    def wait(slot):
        pltpu.make_async_copy(ef_hbm.at[:, pl.ds(0, CHUNK)],
                              ef_v[slot], sems[slot]).wait()
        pltpu.make_async_copy(g_hbm.at[pl.ds(0, CHUNK), :],
                              rows_v[slot], sems[slot]).wait()

    def scatter(c, slot):
        pltpu.async_copy(msg_v[slot],
                         acc_sh.at[dstall_v.at[pl.ds(c * CHUNK, CHUNK)]],
                         ssems[slot], add=True)

    def scatter_wait(slot):
        pltpu.make_async_copy(msg_v[slot], acc_sh.at[pl.ds(0, CHUNK)],
                              ssems[slot]).wait()

    def compute(c, slot):
        erows = rows_v[slot]
        eef = ef_v[slot]
        scatter_wait(slot)

        lanes = lax.iota(jnp.int32, D)

        def edge_body(e, ecarry):
            ef_row = plsc.load_gather(eef, [lanes, jnp.full((D,), e, jnp.int32)])
            bias_t, _ = plsc.unpack(erows[e, pl.ds(256, 32)],
                                    format=plsc.PackFormat.INTERLEAVED)
            terms = [bias_t]
            for p in range(8):
                a, b = plsc.unpack(erows[e, pl.ds(32 * p, 32)],
                                   format=plsc.PackFormat.INTERLEAVED)
                terms.append(ef_row[2 * p] * a + ef_row[2 * p + 1] * b)
            while len(terms) > 1:
                terms = [terms[i] + terms[i + 1]
                         for i in range(0, len(terms) - 1, 2)] + (
                             [terms[-1]] if len(terms) % 2 else [])
            msg_v[slot][e, :] = terms[0]
            return ecarry

        lax.fori_loop(0, CHUNK, edge_body, 0)
        scatter(c, slot)

    issue(0, 0)
    for s in range(2):
        scatter(0, s)

    def pair_body(j, carry):
        c0 = 2 * j
        wait(0)
        issue(c0 + 1, 1)
        compute(c0, 0)
        wait(1)
        issue(c0 + 2, 0)
        compute(c0 + 1, 1)
        return carry

    lax.fori_loop(0, (NCHUNK - 1) // 2, pair_body, 0)
    wait(0)
    compute(NCHUNK - 1, 0)
    scatter_wait(0)
    scatter_wait(1)

    plsc.subcore_barrier()
    pltpu.sync_copy(acc_sh.at[pl.ds(sid * RPT, RPT)],
                    out_hbm.at[cid, pl.ds(sid * RPT, RPT), :])


_sc_call = functools.partial(
    pl.kernel,
    out_type=jax.ShapeDtypeStruct((NC, NPAD, D), jnp.float32),
    mesh=plsc.VectorSubcoreMesh(core_axis_name="c", subcore_axis_name="s"),
    scratch_types=[
        pltpu.VMEM((EPW,), jnp.int32),
        pltpu.VMEM((EPW,), jnp.int32),
        [pltpu.VMEM((D, CHUNK), jnp.float32)] * 2,
        [pltpu.VMEM((CHUNK, DB), jnp.bfloat16)] * 2,
        [pltpu.VMEM((CHUNK, D), jnp.float32)] * 2,
        pltpu.VMEM((RPT, D), jnp.float32),
        pltpu.VMEM_SHARED((NPAD, D), jnp.float32),
        [pltpu.SemaphoreType.DMA] * 2,
        [pltpu.SemaphoreType.DMA] * 2,
    ],
    compiler_params=pltpu.CompilerParams(use_tc_tiling_on_sc=False,
                                         needs_layout_passes=False),
)(_sc_body)


@jax.jit
def kernel(feat, efeat, edge_index, W_edge, b_edge, bias):
    Wr = W_edge.reshape(D, D, D).transpose(1, 0, 2).reshape(D, D * D)
    Waug = jnp.concatenate([Wr, b_edge.reshape(D, D),
                            jnp.zeros((D, DB - DA), jnp.float32)], axis=1)
    Waug = Waug[:, _PERM]

    g_aug = pl.pallas_call(
        _matmul_body,
        grid=(N // MBLK,),
        in_specs=[
            pl.BlockSpec((MBLK, D), lambda i: (i, 0)),
            pl.BlockSpec((D, DB), lambda i: (0, 0)),
        ],
        out_specs=pl.BlockSpec((MBLK, DB), lambda i: (i, 0)),
        out_shape=jax.ShapeDtypeStruct((N, DB), jnp.bfloat16),
    )(feat, Waug)

    partials = _sc_call(g_aug, efeat.T, edge_index[0], edge_index[1])

    out = pl.pallas_call(
        _combine_body,
        out_shape=jax.ShapeDtypeStruct((N, D), jnp.float32),
    )(partials, feat, bias.reshape(1, D))
    return out


# speedup vs baseline: 1.3511x; 1.0252x over previous
import functools

import numpy as np
import jax
import jax.numpy as jnp
from jax import lax
from jax.experimental import pallas as pl
from jax.experimental.pallas import tpu as pltpu
from jax.experimental.pallas import tpu_sc as plsc

N = 10000
E = 160000
D = 16
DA = 272
DB = 288

NC = 2
NS = 16
NW = NC * NS
EPW = E // NW
CHUNK = 200
NCHUNK = EPW // CHUNK
NPAD = 10240
RPT = NPAD // NS

MBLK = 2000


def _matmul_body(f_ref, w_ref, g_ref):
    g_ref[...] = jnp.dot(f_ref[...], w_ref[...],
                         preferred_element_type=jnp.float32
                         ).astype(jnp.bfloat16)


_PERM = np.empty((DB,), np.int32)
for _p in range(DB // 32):
    for _j in range(16):
        _PERM[32 * _p + 2 * _j] = 16 * (2 * _p) + _j
        _PERM[32 * _p + 2 * _j + 1] = 16 * (2 * _p + 1) + _j


def _combine_body(p_ref, f_ref, b_ref, o_ref):
    o_ref[...] = p_ref[0, :N] + p_ref[1, :N] + f_ref[...] + b_ref[...]


def _sc_body(g_hbm, ef_hbm, ei_hbm, out_hbm,
             srcall_v, dstall_v, ef_v, rows_v, msg_v, acc_sh, sems,
             ssems):
    cid = lax.axis_index("c")
    sid = lax.axis_index("s")
    wid = cid * NS + sid
    wbase = wid * EPW

    pltpu.sync_copy(ei_hbm.at[0, pl.ds(wbase, EPW)], srcall_v)
    pltpu.sync_copy(ei_hbm.at[1, pl.ds(wbase, EPW)], dstall_v)

    def zero_msg(r, carry):
        for s in range(3):
            msg_v[s][r, :] = jnp.zeros((D,), jnp.float32)
        return carry

    lax.fori_loop(0, CHUNK, zero_msg, 0)
    for q in range(3):
        pltpu.sync_copy(msg_v[0], acc_sh.at[pl.ds(sid * RPT + q * CHUNK, CHUNK)])
    pltpu.sync_copy(msg_v[0].at[pl.ds(0, RPT - 3 * CHUNK)],
                    acc_sh.at[pl.ds(sid * RPT + 3 * CHUNK, RPT - 3 * CHUNK)])
    plsc.subcore_barrier()

    def issue(c, slot):
        pltpu.async_copy(ef_hbm.at[:, pl.ds(wbase + c * CHUNK, CHUNK)],
                         ef_v[slot], sems[slot])
        pltpu.async_copy(g_hbm.at[srcall_v.at[pl.ds(c * CHUNK, CHUNK)]],
                         rows_v[slot], sems[slot])

    def wait(slot):
        pltpu.make_async_copy(ef_hbm.at[:, pl.ds(0, CHUNK)],
                              ef_v[slot], sems[slot]).wait()
        pltpu.make_async_copy(g_hbm.at[pl.ds(0, CHUNK), :],
                              rows_v[slot], sems[slot]).wait()

    def scatter(c, slot):
        pltpu.async_copy(msg_v[slot],
                         acc_sh.at[dstall_v.at[pl.ds(c * CHUNK, CHUNK)]],
                         ssems[slot], add=True)

    def scatter_wait(slot):
        pltpu.make_async_copy(msg_v[slot], acc_sh.at[pl.ds(0, CHUNK)],
                              ssems[slot]).wait()

    def compute(c, slot):
        erows = rows_v[slot]
        eef = ef_v[slot]
        scatter_wait(slot)

        lanes = lax.iota(jnp.int32, D)

        def edge_body(e, ecarry):
            ef_row = plsc.load_gather(eef, [lanes, jnp.full((D,), e, jnp.int32)])
            bias_t, _ = plsc.unpack(erows[e, pl.ds(256, 32)],
                                    format=plsc.PackFormat.INTERLEAVED)
            terms = [bias_t]
            for p in range(8):
                a, b = plsc.unpack(erows[e, pl.ds(32 * p, 32)],
                                   format=plsc.PackFormat.INTERLEAVED)
                terms.append(ef_row[2 * p] * a + ef_row[2 * p + 1] * b)
            while len(terms) > 1:
                terms = [terms[i] + terms[i + 1]
                         for i in range(0, len(terms) - 1, 2)] + (
                             [terms[-1]] if len(terms) % 2 else [])
            msg_v[slot][e, :] = terms[0]
            return ecarry

        lax.fori_loop(0, CHUNK, edge_body, 0)
        scatter(c, slot)

    issue(0, 0)
    issue(1, 1)
    for s in range(3):
        scatter(0, s)

    def tri_body(j, carry):
        c0 = 3 * j
        for s in range(3):
            wait(s)
            issue(c0 + s + 2, (s + 2) % 3)
            compute(c0 + s, s)
        return carry

    lax.fori_loop(0, 7, tri_body, 0)
    wait(0)
    issue(23, 2)
    compute(21, 0)
    wait(1)
    issue(24, 0)
    compute(22, 1)
    wait(2)
    compute(23, 2)
    wait(0)
    compute(24, 0)
    scatter_wait(0)
    scatter_wait(1)
    scatter_wait(2)

    plsc.subcore_barrier()
    pltpu.sync_copy(acc_sh.at[pl.ds(sid * RPT, RPT)],
                    out_hbm.at[cid, pl.ds(sid * RPT, RPT), :])


_sc_call = functools.partial(
    pl.kernel,
    out_type=jax.ShapeDtypeStruct((NC, NPAD, D), jnp.float32),
    mesh=plsc.VectorSubcoreMesh(core_axis_name="c", subcore_axis_name="s"),
    scratch_types=[
        pltpu.VMEM((EPW,), jnp.int32),
        pltpu.VMEM((EPW,), jnp.int32),
        [pltpu.VMEM((D, CHUNK), jnp.float32)] * 3,
        [pltpu.VMEM((CHUNK, DB), jnp.bfloat16)] * 3,
        [pltpu.VMEM((CHUNK, D), jnp.float32)] * 3,
        pltpu.VMEM_SHARED((NPAD, D), jnp.float32),
        [pltpu.SemaphoreType.DMA] * 3,
        [pltpu.SemaphoreType.DMA] * 3,
    ],
    compiler_params=pltpu.CompilerParams(use_tc_tiling_on_sc=False,
                                         needs_layout_passes=False),
)(_sc_body)


@jax.jit
def kernel(feat, efeat, edge_index, W_edge, b_edge, bias):
    Wr = W_edge.reshape(D, D, D).transpose(1, 0, 2).reshape(D, D * D)
    Waug = jnp.concatenate([Wr, b_edge.reshape(D, D),
                            jnp.zeros((D, DB - DA), jnp.float32)], axis=1)
    Waug = Waug[:, _PERM]

    g_aug = pl.pallas_call(
        _matmul_body,
        grid=(N // MBLK,),
        in_specs=[
            pl.BlockSpec((MBLK, D), lambda i: (i, 0)),
            pl.BlockSpec((D, DB), lambda i: (0, 0)),
        ],
        out_specs=pl.BlockSpec((MBLK, DB), lambda i: (i, 0)),
        out_shape=jax.ShapeDtypeStruct((N, DB), jnp.bfloat16),
    )(feat, Waug)

    partials = _sc_call(g_aug, efeat.T, edge_index)

    out = pl.pallas_call(
        _combine_body,
        out_shape=jax.ShapeDtypeStruct((N, D), jnp.float32),
    )(partials, feat, bias.reshape(1, D))
    return out


# speedup vs baseline: 1.5286x; 1.1314x over previous
import functools

import numpy as np
import jax
import jax.numpy as jnp
from jax import lax
from jax.experimental import pallas as pl
from jax.experimental.pallas import tpu as pltpu
from jax.experimental.pallas import tpu_sc as plsc

N = 10000
E = 160000
D = 16
DA = 272
DB = 288

NC = 2
NS = 16
NW = NC * NS
EPW = E // NW
CHUNK = 200
NCHUNK = EPW // CHUNK
NPAD = 10240
RPT = NPAD // NS

MBLK = 2000


def _round_bf16_bits(x):
    u = jax.lax.bitcast_convert_type(x, jnp.int32)
    return jax.lax.shift_right_logical(
        u + jnp.int32(0x7FFF) + (jax.lax.shift_right_logical(u, 16) & 1), 16)


def _matmul_body(f_ref, we_ref, wo_ref, g_ref):
    ge = jnp.dot(f_ref[...], we_ref[...], preferred_element_type=jnp.float32)
    go = jnp.dot(f_ref[...], wo_ref[...], preferred_element_type=jnp.float32)
    packed = jax.lax.shift_left(_round_bf16_bits(go), 16) | _round_bf16_bits(ge)
    g_ref[...] = jax.lax.bitcast_convert_type(packed, jnp.float32)


def _combine_body(p_ref, f_ref, b_ref, o_ref):
    o_ref[...] = p_ref[0, :N] + p_ref[1, :N] + f_ref[...] + b_ref[...]


def _sc_body(g_hbm, ef_hbm, ei_hbm, out_hbm,
             srcall_v, dstall_v, ef_v, rows_v, msg_v, acc_sh, sems,
             ssems):
    cid = lax.axis_index("c")
    sid = lax.axis_index("s")
    wid = cid * NS + sid
    wbase = wid * EPW

    pltpu.sync_copy(ei_hbm.at[0, pl.ds(wbase, EPW)], srcall_v)
    pltpu.sync_copy(ei_hbm.at[1, pl.ds(wbase, EPW)], dstall_v)

    def zero_msg(r, carry):
        for s in range(3):
            msg_v[s][r, :] = jnp.zeros((D,), jnp.float32)
        return carry

    lax.fori_loop(0, CHUNK, zero_msg, 0)
    for q in range(3):
        pltpu.sync_copy(msg_v[0], acc_sh.at[pl.ds(sid * RPT + q * CHUNK, CHUNK)])
    pltpu.sync_copy(msg_v[0].at[pl.ds(0, RPT - 3 * CHUNK)],
                    acc_sh.at[pl.ds(sid * RPT + 3 * CHUNK, RPT - 3 * CHUNK)])
    plsc.subcore_barrier()

    def issue(c, slot):
        pltpu.async_copy(ef_hbm.at[:, pl.ds(wbase + c * CHUNK, CHUNK)],
                         ef_v[slot], sems[slot])
        pltpu.async_copy(g_hbm.at[srcall_v.at[pl.ds(c * CHUNK, CHUNK)]],
                         rows_v[slot], sems[slot])

    def wait(slot):
        pltpu.make_async_copy(ef_hbm.at[:, pl.ds(0, CHUNK)],
                              ef_v[slot], sems[slot]).wait()
        pltpu.make_async_copy(g_hbm.at[pl.ds(0, CHUNK), :],
                              rows_v[slot], sems[slot]).wait()

    def scatter(c, slot):
        pltpu.async_copy(msg_v[slot],
                         acc_sh.at[dstall_v.at[pl.ds(c * CHUNK, CHUNK)]],
                         ssems[slot], add=True)

    def scatter_wait(slot):
        pltpu.make_async_copy(msg_v[slot], acc_sh.at[pl.ds(0, CHUNK)],
                              ssems[slot]).wait()

    def compute(c, slot):
        erows = rows_v[slot]
        eef = ef_v[slot]
        scatter_wait(slot)

        lanes = lax.iota(jnp.int32, D)

        def edge_body(e, ecarry):
            ef_row = plsc.load_gather(eef, [lanes, jnp.full((D,), e, jnp.int32)])
            def halves(p):
                xi = plsc.bitcast(erows[e, pl.ds(16 * p, 16)], jnp.int32)
                a = plsc.bitcast(xi << 16, jnp.float32)
                b = plsc.bitcast(xi & jnp.int32(-65536), jnp.float32)
                return a, b

            bias_t, _ = halves(8)
            terms = [bias_t]
            for p in range(8):
                a, b = halves(p)
                terms.append(ef_row[2 * p] * a + ef_row[2 * p + 1] * b)
            while len(terms) > 1:
                terms = [terms[i] + terms[i + 1]
                         for i in range(0, len(terms) - 1, 2)] + (
                             [terms[-1]] if len(terms) % 2 else [])
            msg_v[slot][e, :] = terms[0]
            return ecarry

        lax.fori_loop(0, CHUNK, edge_body, 0)
        scatter(c, slot)

    issue(0, 0)
    issue(1, 1)
    for s in range(3):
        scatter(0, s)

    def tri_body(j, carry):
        c0 = 3 * j
        for s in range(3):
            wait(s)
            issue(c0 + s + 2, (s + 2) % 3)
            compute(c0 + s, s)
        return carry

    lax.fori_loop(0, 7, tri_body, 0)
    wait(0)
    issue(23, 2)
    compute(21, 0)
    wait(1)
    issue(24, 0)
    compute(22, 1)
    wait(2)
    compute(23, 2)
    wait(0)
    compute(24, 0)
    scatter_wait(0)
    scatter_wait(1)
    scatter_wait(2)

    plsc.subcore_barrier()
    pltpu.sync_copy(acc_sh.at[pl.ds(sid * RPT, RPT)],
                    out_hbm.at[cid, pl.ds(sid * RPT, RPT), :])


_sc_call = functools.partial(
    pl.kernel,
    out_type=jax.ShapeDtypeStruct((NC, NPAD, D), jnp.float32),
    mesh=plsc.VectorSubcoreMesh(core_axis_name="c", subcore_axis_name="s"),
    scratch_types=[
        pltpu.VMEM((EPW,), jnp.int32),
        pltpu.VMEM((EPW,), jnp.int32),
        [pltpu.VMEM((D, CHUNK), jnp.float32)] * 3,
        [pltpu.VMEM((CHUNK, DB // 2), jnp.float32)] * 3,
        [pltpu.VMEM((CHUNK, D), jnp.float32)] * 3,
        pltpu.VMEM_SHARED((NPAD, D), jnp.float32),
        [pltpu.SemaphoreType.DMA] * 3,
        [pltpu.SemaphoreType.DMA] * 3,
    ],
    compiler_params=pltpu.CompilerParams(use_tc_tiling_on_sc=False,
                                         needs_layout_passes=False),
)(_sc_body)


@jax.jit
def kernel(feat, efeat, edge_index, W_edge, b_edge, bias):
    Wr = W_edge.reshape(D, D, D).transpose(1, 0, 2).reshape(D, D * D)
    Waug = jnp.concatenate([Wr, b_edge.reshape(D, D),
                            jnp.zeros((D, DB - DA), jnp.float32)], axis=1)
    Wb = Waug.reshape(D, DB // D, D)
    We = Wb[:, 0::2, :].reshape(D, DB // 2)
    Wo = Wb[:, 1::2, :].reshape(D, DB // 2)

    g_aug = pl.pallas_call(
        _matmul_body,
        grid=(N // MBLK,),
        in_specs=[
            pl.BlockSpec((MBLK, D), lambda i: (i, 0)),
            pl.BlockSpec((D, DB // 2), lambda i: (0, 0)),
            pl.BlockSpec((D, DB // 2), lambda i: (0, 0)),
        ],
        out_specs=pl.BlockSpec((MBLK, DB // 2), lambda i: (i, 0)),
        out_shape=jax.ShapeDtypeStruct((N, DB // 2), jnp.float32),
    )(feat, We, Wo)

    partials = _sc_call(g_aug, efeat.T, edge_index)

    out = pl.pallas_call(
        _combine_body,
        out_shape=jax.ShapeDtypeStruct((N, D), jnp.float32),
    )(partials, feat, bias.reshape(1, D))
    return out


# speedup vs baseline: 1.5299x; 1.0008x over previous
import functools

import jax
import jax.numpy as jnp
from jax import lax
from jax.experimental import pallas as pl
from jax.experimental.pallas import tpu as pltpu
from jax.experimental.pallas import tpu_sc as plsc

N = 10000
E = 160000
D = 16
DA = 272
DB = 288

NC = 2
NS = 16
NW = NC * NS
EPW = E // NW
CHUNK = 200
NCHUNK = EPW // CHUNK
NPAD = 10240
RPT = NPAD // NS

MBLK = 2000


def _round_bf16_bits(x):
    u = jax.lax.bitcast_convert_type(x, jnp.int32)
    return jax.lax.shift_right_logical(
        u + jnp.int32(0x7FFF) + (jax.lax.shift_right_logical(u, 16) & 1), 16)


def _matmul_body(f_ref, we_ref, wo_ref, g_ref):
    ge = jnp.dot(f_ref[...], we_ref[...], preferred_element_type=jnp.float32)
    go = jnp.dot(f_ref[...], wo_ref[...], preferred_element_type=jnp.float32)
    packed = jax.lax.shift_left(_round_bf16_bits(go), 16) | _round_bf16_bits(ge)
    g_ref[...] = jax.lax.bitcast_convert_type(packed, jnp.float32)


def _combine_body(p_ref, f_ref, b_ref, o_ref):
    o_ref[...] = p_ref[0, :N] + p_ref[1, :N] + f_ref[...] + b_ref[...]


def _sc_body(g_hbm, ef_hbm, ei_hbm, out_hbm,
             srcall_v, dstall_v, ef_v, rows_v, msg_v, acc_sh, sems,
             ssems):
    cid = lax.axis_index("c")
    sid = lax.axis_index("s")
    wid = cid * NS + sid
    wbase = wid * EPW

    pltpu.sync_copy(ei_hbm.at[0, pl.ds(wbase, EPW)], srcall_v)
    pltpu.sync_copy(ei_hbm.at[1, pl.ds(wbase, EPW)], dstall_v)

    def zero_msg(r, carry):
        for s in range(3):
            msg_v[s][r, :] = jnp.zeros((D,), jnp.float32)
        return carry

    lax.fori_loop(0, CHUNK, zero_msg, 0)
    for q in range(3):
        pltpu.sync_copy(msg_v[0], acc_sh.at[pl.ds(sid * RPT + q * CHUNK, CHUNK)])
    pltpu.sync_copy(msg_v[0].at[pl.ds(0, RPT - 3 * CHUNK)],
                    acc_sh.at[pl.ds(sid * RPT + 3 * CHUNK, RPT - 3 * CHUNK)])
    plsc.subcore_barrier()

    def issue(c, slot):
        pltpu.async_copy(ef_hbm.at[:, pl.ds(wbase + c * CHUNK, CHUNK)],
                         ef_v[slot], sems[slot])
        pltpu.async_copy(g_hbm.at[srcall_v.at[pl.ds(c * CHUNK, CHUNK)]],
                         rows_v[slot], sems[slot])

    def wait(slot):
        pltpu.make_async_copy(ef_hbm.at[:, pl.ds(0, CHUNK)],
                              ef_v[slot], sems[slot]).wait()
        pltpu.make_async_copy(g_hbm.at[pl.ds(0, CHUNK), :],
                              rows_v[slot], sems[slot]).wait()

    def scatter(c, slot):
        pltpu.async_copy(msg_v[slot],
                         acc_sh.at[dstall_v.at[pl.ds(c * CHUNK, CHUNK)]],
                         ssems[slot], add=True)

    def scatter_wait(slot):
        pltpu.make_async_copy(msg_v[slot], acc_sh.at[pl.ds(0, CHUNK)],
                              ssems[slot]).wait()

    def compute(c, slot):
        erows = rows_v[slot]
        eef = ef_v[slot]
        scatter_wait(slot)

        lanes = lax.iota(jnp.int32, D)

        def edge_body(e, ecarry):
            ef_row = plsc.load_gather(eef, [lanes, jnp.full((D,), e, jnp.int32)])
            def halves(p):
                xi = plsc.bitcast(erows[e, pl.ds(16 * p, 16)], jnp.int32)
                a = plsc.bitcast(xi << 16, jnp.float32)
                b = plsc.bitcast(xi & jnp.int32(-65536), jnp.float32)
                return a, b

            bias_t, _ = halves(8)
            terms = [bias_t]
            for p in range(8):
                a, b = halves(p)
                terms.append(ef_row[2 * p] * a + ef_row[2 * p + 1] * b)
            while len(terms) > 1:
                terms = [terms[i] + terms[i + 1]
                         for i in range(0, len(terms) - 1, 2)] + (
                             [terms[-1]] if len(terms) % 2 else [])
            msg_v[slot][e, :] = terms[0]
            return ecarry

        lax.fori_loop(0, CHUNK, edge_body, 0)
        scatter(c, slot)

    issue(0, 0)
    issue(1, 1)
    for s in range(3):
        scatter(0, s)

    def tri_body(j, carry):
        c0 = 3 * j
        for s in range(3):
            wait(s)
            issue(c0 + s + 2, (s + 2) % 3)
            compute(c0 + s, s)
        return carry

    lax.fori_loop(0, 7, tri_body, 0)
    wait(0)
    issue(23, 2)
    compute(21, 0)
    wait(1)
    issue(24, 0)
    compute(22, 1)
    wait(2)
    compute(23, 2)
    wait(0)
    compute(24, 0)
    scatter_wait(0)
    scatter_wait(1)
    scatter_wait(2)

    plsc.subcore_barrier()
    pltpu.sync_copy(acc_sh.at[pl.ds(sid * RPT, RPT)],
                    out_hbm.at[cid, pl.ds(sid * RPT, RPT), :])


_sc_call = functools.partial(
    pl.kernel,
    out_type=jax.ShapeDtypeStruct((NC, NPAD, D), jnp.float32),
    mesh=plsc.VectorSubcoreMesh(core_axis_name="c", subcore_axis_name="s"),
    scratch_types=[
        pltpu.VMEM((EPW,), jnp.int32),
        pltpu.VMEM((EPW,), jnp.int32),
        [pltpu.VMEM((D, CHUNK), jnp.float32)] * 3,
        [pltpu.VMEM((CHUNK, DB // 2), jnp.float32)] * 3,
        [pltpu.VMEM((CHUNK, D), jnp.float32)] * 3,
        pltpu.VMEM_SHARED((NPAD, D), jnp.float32),
        [pltpu.SemaphoreType.DMA] * 3,
        [pltpu.SemaphoreType.DMA] * 3,
    ],
    compiler_params=pltpu.CompilerParams(use_tc_tiling_on_sc=False,
                                         needs_layout_passes=False),
)(_sc_body)


@jax.jit
def kernel(feat, efeat, edge_index, W_edge, b_edge, bias):
    Wr = W_edge.reshape(D, D, D).transpose(1, 0, 2).reshape(D, D * D)
    Waug = jnp.concatenate([Wr, b_edge.reshape(D, D),
                            jnp.zeros((D, DB - DA), jnp.float32)], axis=1)
    Wb = Waug.reshape(D, DB // D, D)
    We = Wb[:, 0::2, :].reshape(D, DB // 2)
    Wo = Wb[:, 1::2, :].reshape(D, DB // 2)

    g_aug = pl.pallas_call(
        _matmul_body,
        grid=(N // MBLK,),
        in_specs=[
            pl.BlockSpec((MBLK, D), lambda i: (i, 0)),
            pl.BlockSpec((D, DB // 2), lambda i: (0, 0)),
            pl.BlockSpec((D, DB // 2), lambda i: (0, 0)),
        ],
        out_specs=pl.BlockSpec((MBLK, DB // 2), lambda i: (i, 0)),
        out_shape=jax.ShapeDtypeStruct((N, DB // 2), jnp.float32),
    )(feat, We, Wo)

    partials = _sc_call(g_aug, efeat.T, edge_index)

    out = pl.pallas_call(
        _combine_body,
        out_shape=jax.ShapeDtypeStruct((N, D), jnp.float32),
    )(partials, feat, bias.reshape(1, D))
    return out
